# baseline (speedup 1.0000x reference)
import functools

import jax
import jax.numpy as jnp
from jax import lax
from jax.experimental import pallas as pl
from jax.experimental.pallas import tpu as pltpu
from jax.experimental.pallas import tpu_sc as plsc

ALPHA = 0.1
KHOPS = 10

B = 64
GRP = 2
NSC = 2
NSUB = 16


def _eprep_body(nh, dst_ref, out_ref):
    c = pl.program_id(0)
    d = dst_ref[...]
    local = d - c * nh
    ok = (local >= 0) & (local < nh)
    out_ref[...] = jnp.where(ok, local, nh)[None]


def _eprep(dstp, nh):
    nchunk = dstp.shape[0]
    eb = 512
    assert nchunk % eb == 0
    grid = (NSC, nchunk // eb)
    return pl.pallas_call(
        functools.partial(_eprep_body, nh),
        grid=grid,
        in_specs=[pl.BlockSpec((eb, B), lambda c, g: (g, 0))],
        out_specs=pl.BlockSpec((1, eb, B), lambda c, g: (c, g, 0)),
        out_shape=jax.ShapeDtypeStruct((NSC, nchunk, B), jnp.int32),
    )(dstp)


def _deg_body(n, nchunk, dst_hbm, out_hbm, degv, didxb):
    c = lax.axis_index("c")
    s = lax.axis_index("s")
    w = c * NSUB + s
    zero16 = jnp.zeros((16,), jnp.float32)

    def zbody(i, _):
        degv[pl.ds(i * 16, 16)] = zero16
        return _

    lax.fori_loop(0, n // 16, zbody, None)

    ones16 = jnp.ones((16,), jnp.float32)
    gpt = nchunk // (NSC * NSUB) // GRP

    def gbody(g, _):
        g0 = (w * gpt + g) * GRP
        pltpu.sync_copy(dst_hbm.at[pl.ds(g0, GRP)], didxb)
        for jj in range(GRP):
            for ii in range(B // 16):
                idx = didxb[jj, pl.ds(ii * 16, 16)]
                plsc.addupdate_scatter(degv, [idx], ones16, mask=idx >= 0)
        return _

    lax.fori_loop(0, gpt, gbody, None)
    pltpu.sync_copy(degv, out_hbm.at[w])


def _deg(dstp, n):
    nchunk = dstp.shape[0]
    mesh = plsc.VectorSubcoreMesh(core_axis_name="c", subcore_axis_name="s")
    return pl.kernel(
        functools.partial(_deg_body, n, nchunk),
        out_type=jax.ShapeDtypeStruct((NSC * NSUB, n), jnp.float32),
        mesh=mesh,
        scratch_types=[
            pltpu.VMEM((n,), jnp.float32),
            pltpu.VMEM((GRP, B), jnp.int32),
        ],
        compiler_params=pltpu.CompilerParams(
            needs_layout_passes=False, use_tc_tiling_on_sc=False),
    )(dstp)


def _prep_body(c, part_ref, x_ref, w_ref, b_ref,
               u0_ref, cc_ref, aa_ref, ah_ref, dis_ref):
    deg = jnp.sum(part_ref[...], axis=1)
    h = jnp.dot(x_ref[...], w_ref[...],
                preferred_element_type=jnp.float32,
                precision=lax.Precision.HIGHEST) + b_ref[...]
    dis = jnp.where(deg > 0, lax.rsqrt(jnp.maximum(deg, 1e-12)), 0.0)
    d1 = dis[:, None]
    r = h.shape[0]
    u0_ref[...] = d1 * h
    aa_ref[...] = ALPHA * (d1 * h)
    cc_ref[...] = jnp.broadcast_to((1.0 - ALPHA) * d1 * d1, (r, c))
    ah_ref[...] = ALPHA * h
    dis_ref[...] = jnp.broadcast_to(d1, (r, 8))


def _prep(partials, x, w, b2, n, c):
    r = 1000
    f = x.shape[1]
    nw = partials.shape[1]
    grid = (n // r,)
    outs = [
        jax.ShapeDtypeStruct((n, c), jnp.float32),
        jax.ShapeDtypeStruct((n, c), jnp.float32),
        jax.ShapeDtypeStruct((n, c), jnp.float32),
        jax.ShapeDtypeStruct((n, c), jnp.float32),
        jax.ShapeDtypeStruct((n, 8), jnp.float32),
    ]
    return pl.pallas_call(
        functools.partial(_prep_body, c),
        grid=grid,
        in_specs=[
            pl.BlockSpec((r, nw), lambda g: (g, 0)),
            pl.BlockSpec((r, f), lambda g: (g, 0)),
            pl.BlockSpec((f, c), lambda g: (0, 0)),
            pl.BlockSpec((1, c), lambda g: (0, 0)),
        ],
        out_specs=[
            pl.BlockSpec((r, c), lambda g: (g, 0)),
            pl.BlockSpec((r, c), lambda g: (g, 0)),
            pl.BlockSpec((r, c), lambda g: (g, 0)),
            pl.BlockSpec((r, c), lambda g: (g, 0)),
            pl.BlockSpec((r, 8), lambda g: (g, 0)),
        ],
        out_shape=outs,
    )(partials, x, w, b2)


def _hop_body(n, nh, rt, nchunk, c_feat,
              u_hbm, src_hbm, dstl_hbm, zr_hbm, out_hbm,
              acc, sidx0, sidx1, didx0, didx1, rows0, rows1,
              gsem0, gsem1, ssem0, ssem1, is_s0, is_s1, is_d0, is_d1):
    c = lax.axis_index("c")
    s = lax.axis_index("s")

    pltpu.sync_copy(zr_hbm, acc.at[pl.ds(s * rt, rt)])

    @pl.when(s == NSUB - 1)
    def _():
        pltpu.sync_copy(zr_hbm.at[pl.ds(0, 8)], acc.at[pl.ds(nh, 8)])

    plsc.subcore_barrier()

    cpt = nchunk // NSUB
    base = s * cpt
    dbase = c * nchunk + base
    ngroups = cpt // GRP
    numu = ngroups // 2

    pltpu.sync_copy(src_hbm.at[pl.ds(base, GRP)], sidx0)
    pltpu.sync_copy(dstl_hbm.at[pl.ds(dbase, GRP)], didx0)

    def body(u_, _):
        k0 = u_ * 2 * GRP
        ib_s = pltpu.async_copy(
            src_hbm.at[pl.ds(base + k0 + GRP, GRP)], sidx1, is_s1)
        ib_d = pltpu.async_copy(
            dstl_hbm.at[pl.ds(dbase + k0 + GRP, GRP)], didx1, is_d1)
        g0 = pltpu.async_copy(u_hbm.at[sidx0.at[0]], rows0, gsem0)
        g1 = pltpu.async_copy(u_hbm.at[sidx0.at[1]], rows1, gsem1)
        g0.wait()
        s0 = pltpu.async_copy(rows0, acc.at[didx0.at[0]], ssem0, add=True)
        g1.wait()
        ib_s.wait()
        ib_d.wait()
        s0.wait()
        g2 = pltpu.async_copy(u_hbm.at[sidx1.at[0]], rows0, gsem0)
        s1 = pltpu.async_copy(rows1, acc.at[didx0.at[1]], ssem1, add=True)
        g2.wait()
        s1.wait()
        soff = jnp.minimum(base + k0 + 2 * GRP, nchunk - GRP)
        doff = jnp.minimum(dbase + k0 + 2 * GRP, NSC * nchunk - GRP)
        ia_s = pltpu.async_copy(src_hbm.at[pl.ds(soff, GRP)], sidx0, is_s0)
        ia_d = pltpu.async_copy(dstl_hbm.at[pl.ds(doff, GRP)], didx0, is_d0)
        g3 = pltpu.async_copy(u_hbm.at[sidx1.at[1]], rows1, gsem1)
        s2 = pltpu.async_copy(rows0, acc.at[didx1.at[0]], ssem0, add=True)
        g3.wait()
        s2.wait()
        s3 = pltpu.async_copy(rows1, acc.at[didx1.at[1]], ssem1, add=True)
        s3.wait()
        ia_s.wait()
        ia_d.wait()
        return _

    lax.fori_loop(0, numu, body, None)
    plsc.subcore_barrier()

    w = c * NSUB + s
    pltpu.sync_copy(acc.at[pl.ds(s * rt, rt)], out_hbm.at[w])


def _hop(u, srcp, dstl, zr, n, nh, rt, c):
    nchunk = srcp.shape[0]
    mesh = plsc.VectorSubcoreMesh(core_axis_name="c", subcore_axis_name="s")
    return pl.kernel(
        functools.partial(_hop_body, n, nh, rt, nchunk, c),
        out_type=jax.ShapeDtypeStruct((NSC * NSUB, rt, c), jnp.float32),
        mesh=mesh,
        scratch_types=[
            pltpu.VMEM_SHARED((nh + 8, c), jnp.float32),
            pltpu.VMEM((GRP, B), jnp.int32),
            pltpu.VMEM((GRP, B), jnp.int32),
            pltpu.VMEM((GRP, B), jnp.int32),
            pltpu.VMEM((GRP, B), jnp.int32),
            pltpu.VMEM((B, c), jnp.float32),
            pltpu.VMEM((B, c), jnp.float32),
            pltpu.SemaphoreType.DMA,
            pltpu.SemaphoreType.DMA,
            pltpu.SemaphoreType.DMA,
            pltpu.SemaphoreType.DMA,
            pltpu.SemaphoreType.DMA,
            pltpu.SemaphoreType.DMA,
            pltpu.SemaphoreType.DMA,
            pltpu.SemaphoreType.DMA,
        ],
        compiler_params=pltpu.CompilerParams(
            needs_layout_passes=False, use_tc_tiling_on_sc=False),
    )(u, srcp, dstl, zr)


def _upd_body(raw_ref, cc_ref, aa_ref, out_ref):
    out_ref[...] = raw_ref[...] * cc_ref[...] + aa_ref[...]


def _upd(raw, cc, aa, n, c):
    r = 2000
    grid = (n // r,)
    spec = pl.BlockSpec((r, c), lambda g: (g, 0))
    return pl.pallas_call(
        _upd_body,
        grid=grid,
        in_specs=[spec, spec, spec],
        out_specs=spec,
        out_shape=jax.ShapeDtypeStruct((n, c), jnp.float32),
    )(raw, cc, aa)


def _final_body(raw_ref, dis_ref, ah_ref, out_ref):
    z = (1.0 - ALPHA) * dis_ref[:, 0:1] * raw_ref[...] + ah_ref[...]
    m = jnp.max(z, axis=1, keepdims=True)
    lse = jnp.log(jnp.sum(jnp.exp(z - m), axis=1, keepdims=True))
    out_ref[...] = z - m - lse


def _final(raw, dis8, ah, n, c):
    r = 2000
    grid = (n // r,)
    spec = pl.BlockSpec((r, c), lambda g: (g, 0))
    return pl.pallas_call(
        _final_body,
        grid=grid,
        in_specs=[spec, pl.BlockSpec((r, 8), lambda g: (g, 0)), spec],
        out_specs=spec,
        out_shape=jax.ShapeDtypeStruct((n, c), jnp.float32),
    )(raw, dis8, ah)


def kernel(x, edge_index, W, b):
    n, f = x.shape
    c = W.shape[1]
    e = edge_index.shape[1]
    nh = n // 2
    rt = nh // NSUB

    src = edge_index[0]
    dst = edge_index[1]

    quant = 512 * B
    epad = (-e) % quant
    if epad:
        src = jnp.concatenate([src, jnp.zeros((epad,), jnp.int32)])
        dst = jnp.concatenate([dst, jnp.full((epad,), -1, jnp.int32)])
    nchunk = (e + epad) // B
    srcp = src.reshape(nchunk, B)
    dstp = dst.reshape(nchunk, B)

    dstl = _eprep(dstp, nh).reshape(NSC * nchunk, B)
    partials = _deg(dstp, n)
    b2 = b.reshape(1, c)
    u0, cc, aa, ah, dis8 = _prep(partials.T, x, W, b2, n, c)

    zr = jnp.zeros((rt, c), jnp.float32)
    u = u0
    raw = None
    for k in range(KHOPS):
        raw = _hop(u, srcp, dstl, zr, n, nh, rt, c).reshape(n, c)
        if k < KHOPS - 1:
            u = _upd(raw, cc, aa, n, c)
    return _final(raw, dis8, ah, n, c)


# speedup vs baseline: 1.0910x; 1.0910x over previous
import functools

import jax
import jax.numpy as jnp
from jax import lax
from jax.experimental import pallas as pl
from jax.experimental.pallas import tpu as pltpu
from jax.experimental.pallas import tpu_sc as plsc

ALPHA = 0.1
KHOPS = 10

B = 64
GRP = 2
NSC = 2
NSUB = 16


def _eprep_body(nh, dst_ref, out_ref):
    c = pl.program_id(0)
    d = dst_ref[...]
    local = d - c * nh
    ok = (local >= 0) & (local < nh)
    out_ref[...] = jnp.where(ok, local, nh)[None]


def _eprep(dstp, nh):
    nchunk = dstp.shape[0]
    eb = 64
    assert nchunk % eb == 0
    grid = (NSC, nchunk // eb)
    return pl.pallas_call(
        functools.partial(_eprep_body, nh),
        grid=grid,
        in_specs=[pl.BlockSpec((eb, B), lambda c, g: (g, 0))],
        out_specs=pl.BlockSpec((1, eb, B), lambda c, g: (c, g, 0)),
        out_shape=jax.ShapeDtypeStruct((NSC, nchunk, B), jnp.int32),
    )(dstp)


def _deg_body(n, nchunk, dst_hbm, out_hbm, degv, didxb):
    c = lax.axis_index("c")
    s = lax.axis_index("s")
    w = c * NSUB + s
    zero16 = jnp.zeros((16,), jnp.float32)

    def zbody(i, _):
        degv[pl.ds(i * 16, 16)] = zero16
        return _

    lax.fori_loop(0, n // 16, zbody, None)

    ones16 = jnp.ones((16,), jnp.float32)
    gpt = nchunk // (NSC * NSUB) // GRP

    def gbody(g, _):
        g0 = (w * gpt + g) * GRP
        pltpu.sync_copy(dst_hbm.at[pl.ds(g0, GRP)], didxb)
        for jj in range(GRP):
            for ii in range(B // 16):
                idx = didxb[jj, pl.ds(ii * 16, 16)]
                plsc.addupdate_scatter(degv, [idx], ones16, mask=idx >= 0)
        return _

    lax.fori_loop(0, gpt, gbody, None)
    pltpu.sync_copy(degv, out_hbm.at[w])


def _deg(dstp, n):
    nchunk = dstp.shape[0]
    mesh = plsc.VectorSubcoreMesh(core_axis_name="c", subcore_axis_name="s")
    return pl.kernel(
        functools.partial(_deg_body, n, nchunk),
        out_type=jax.ShapeDtypeStruct((NSC * NSUB, n), jnp.float32),
        mesh=mesh,
        scratch_types=[
            pltpu.VMEM((n,), jnp.float32),
            pltpu.VMEM((GRP, B), jnp.int32),
        ],
        compiler_params=pltpu.CompilerParams(
            needs_layout_passes=False, use_tc_tiling_on_sc=False),
    )(dstp)


def _prep_body(c, part_ref, x_ref, w_ref, b_ref,
               u0_ref, cc_ref, aa_ref, ah_ref, dis_ref):
    deg = jnp.sum(part_ref[...], axis=1)
    h = jnp.dot(x_ref[...], w_ref[...],
                preferred_element_type=jnp.float32,
                precision=lax.Precision.HIGHEST) + b_ref[...]
    dis = jnp.where(deg > 0, lax.rsqrt(jnp.maximum(deg, 1e-12)), 0.0)
    d1 = dis[:, None]
    r = h.shape[0]
    u0_ref[...] = d1 * h
    aa_ref[...] = ALPHA * (d1 * h)
    cc_ref[...] = jnp.broadcast_to((1.0 - ALPHA) * d1 * d1, (r, c))
    ah_ref[...] = ALPHA * h
    dis_ref[...] = jnp.broadcast_to(d1, (r, 8))


def _prep(partials, x, w, b2, n, c):
    r = 1000
    f = x.shape[1]
    nw = partials.shape[1]
    grid = (n // r,)
    outs = [
        jax.ShapeDtypeStruct((n, c), jnp.float32),
        jax.ShapeDtypeStruct((n, c), jnp.float32),
        jax.ShapeDtypeStruct((n, c), jnp.float32),
        jax.ShapeDtypeStruct((n, c), jnp.float32),
        jax.ShapeDtypeStruct((n, 8), jnp.float32),
    ]
    return pl.pallas_call(
        functools.partial(_prep_body, c),
        grid=grid,
        in_specs=[
            pl.BlockSpec((r, nw), lambda g: (g, 0)),
            pl.BlockSpec((r, f), lambda g: (g, 0)),
            pl.BlockSpec((f, c), lambda g: (0, 0)),
            pl.BlockSpec((1, c), lambda g: (0, 0)),
        ],
        out_specs=[
            pl.BlockSpec((r, c), lambda g: (g, 0)),
            pl.BlockSpec((r, c), lambda g: (g, 0)),
            pl.BlockSpec((r, c), lambda g: (g, 0)),
            pl.BlockSpec((r, c), lambda g: (g, 0)),
            pl.BlockSpec((r, 8), lambda g: (g, 0)),
        ],
        out_shape=outs,
    )(partials, x, w, b2)


def _hop_body(n, nh, rt, nchunk, c_feat,
              u_hbm, src_hbm, dstl_hbm, zr_hbm, out_hbm,
              acc, sidx0, sidx1, didx0, didx1, rows0, rows1,
              gsem0, gsem1, ssem0, ssem1, is_s0, is_s1, is_d0, is_d1):
    c = lax.axis_index("c")
    s = lax.axis_index("s")

    pltpu.sync_copy(zr_hbm, acc.at[pl.ds(s * rt, rt)])

    @pl.when(s == NSUB - 1)
    def _():
        pltpu.sync_copy(zr_hbm.at[pl.ds(0, 8)], acc.at[pl.ds(nh, 8)])

    plsc.subcore_barrier()

    cpt = nchunk // NSUB
    base = s * cpt
    dbase = c * nchunk + base
    ngroups = cpt // GRP
    numu = ngroups // 2

    pltpu.sync_copy(src_hbm.at[pl.ds(base, GRP)], sidx0)
    pltpu.sync_copy(dstl_hbm.at[pl.ds(dbase, GRP)], didx0)

    def body(u_, _):
        k0 = u_ * 2 * GRP
        ib_s = pltpu.async_copy(
            src_hbm.at[pl.ds(base + k0 + GRP, GRP)], sidx1, is_s1)
        ib_d = pltpu.async_copy(
            dstl_hbm.at[pl.ds(dbase + k0 + GRP, GRP)], didx1, is_d1)
        g0 = pltpu.async_copy(u_hbm.at[sidx0.at[0]], rows0, gsem0)
        g1 = pltpu.async_copy(u_hbm.at[sidx0.at[1]], rows1, gsem1)
        g0.wait()
        s0 = pltpu.async_copy(rows0, acc.at[didx0.at[0]], ssem0, add=True)
        g1.wait()
        ib_s.wait()
        ib_d.wait()
        s0.wait()
        g2 = pltpu.async_copy(u_hbm.at[sidx1.at[0]], rows0, gsem0)
        s1 = pltpu.async_copy(rows1, acc.at[didx0.at[1]], ssem1, add=True)
        g2.wait()
        s1.wait()
        soff = jnp.minimum(base + k0 + 2 * GRP, nchunk - GRP)
        doff = jnp.minimum(dbase + k0 + 2 * GRP, NSC * nchunk - GRP)
        ia_s = pltpu.async_copy(src_hbm.at[pl.ds(soff, GRP)], sidx0, is_s0)
        ia_d = pltpu.async_copy(dstl_hbm.at[pl.ds(doff, GRP)], didx0, is_d0)
        g3 = pltpu.async_copy(u_hbm.at[sidx1.at[1]], rows1, gsem1)
        s2 = pltpu.async_copy(rows0, acc.at[didx1.at[0]], ssem0, add=True)
        g3.wait()
        s2.wait()
        s3 = pltpu.async_copy(rows1, acc.at[didx1.at[1]], ssem1, add=True)
        s3.wait()
        ia_s.wait()
        ia_d.wait()
        return _

    lax.fori_loop(0, numu, body, None)
    plsc.subcore_barrier()

    w = c * NSUB + s
    pltpu.sync_copy(acc.at[pl.ds(s * rt, rt)], out_hbm.at[w])


def _hop(u, srcp, dstl, zr, n, nh, rt, c):
    nchunk = srcp.shape[0]
    mesh = plsc.VectorSubcoreMesh(core_axis_name="c", subcore_axis_name="s")
    return pl.kernel(
        functools.partial(_hop_body, n, nh, rt, nchunk, c),
        out_type=jax.ShapeDtypeStruct((NSC * NSUB, rt, c), jnp.float32),
        mesh=mesh,
        scratch_types=[
            pltpu.VMEM_SHARED((nh + 8, c), jnp.float32),
            pltpu.VMEM((GRP, B), jnp.int32),
            pltpu.VMEM((GRP, B), jnp.int32),
            pltpu.VMEM((GRP, B), jnp.int32),
            pltpu.VMEM((GRP, B), jnp.int32),
            pltpu.VMEM((B, c), jnp.float32),
            pltpu.VMEM((B, c), jnp.float32),
            pltpu.SemaphoreType.DMA,
            pltpu.SemaphoreType.DMA,
            pltpu.SemaphoreType.DMA,
            pltpu.SemaphoreType.DMA,
            pltpu.SemaphoreType.DMA,
            pltpu.SemaphoreType.DMA,
            pltpu.SemaphoreType.DMA,
            pltpu.SemaphoreType.DMA,
        ],
        compiler_params=pltpu.CompilerParams(
            needs_layout_passes=False, use_tc_tiling_on_sc=False),
    )(u, srcp, dstl, zr)


def _upd_body(flen, raw_hbm, cc_hbm, aa_hbm, out_hbm, rbuf, cbuf, abuf):
    c = lax.axis_index("c")
    s = lax.axis_index("s")
    w = c * NSUB + s
    ch = rbuf.shape[0]
    nfull = flen // ch
    rem = flen - nfull * ch

    def run_chunk(off, size):
        pltpu.sync_copy(raw_hbm.at[w, pl.ds(off, size)], rbuf.at[pl.ds(0, size)])
        pltpu.sync_copy(cc_hbm.at[w, pl.ds(off, size)], cbuf.at[pl.ds(0, size)])
        pltpu.sync_copy(aa_hbm.at[w, pl.ds(off, size)], abuf.at[pl.ds(0, size)])

        def vbody(i, t):
            p = i * 16
            rbuf[pl.ds(p, 16)] = (rbuf[pl.ds(p, 16)] * cbuf[pl.ds(p, 16)]
                                  + abuf[pl.ds(p, 16)])
            return t

        if size % 16:
            p = size - 16
            tailv = (rbuf[pl.ds(p, 16)] * cbuf[pl.ds(p, 16)]
                     + abuf[pl.ds(p, 16)])
            tailv = lax.fori_loop(0, size // 16, vbody, tailv)
            rbuf[pl.ds(p, 16)] = tailv
        else:
            lax.fori_loop(0, size // 16, vbody, jnp.zeros((16,), jnp.float32))
        pltpu.sync_copy(rbuf.at[pl.ds(0, size)], out_hbm.at[w, pl.ds(off, size)])

    def cbody(j, _):
        run_chunk(j * ch, ch)
        return _

    lax.fori_loop(0, nfull, cbody, None)
    if rem:
        run_chunk(nfull * ch, rem)


def _upd(raw32, cc32, aa32, nw, flen):
    ch = 24576
    mesh = plsc.VectorSubcoreMesh(core_axis_name="c", subcore_axis_name="s")
    return pl.kernel(
        functools.partial(_upd_body, flen),
        out_type=jax.ShapeDtypeStruct((nw, flen), jnp.float32),
        mesh=mesh,
        scratch_types=[
            pltpu.VMEM((ch,), jnp.float32),
            pltpu.VMEM((ch,), jnp.float32),
            pltpu.VMEM((ch,), jnp.float32),
        ],
        compiler_params=pltpu.CompilerParams(
            needs_layout_passes=False, use_tc_tiling_on_sc=False),
    )(raw32, cc32, aa32)


def _final_body(raw_ref, dis_ref, ah_ref, out_ref):
    z = (1.0 - ALPHA) * dis_ref[:, 0:1] * raw_ref[...] + ah_ref[...]
    m = jnp.max(z, axis=1, keepdims=True)
    lse = jnp.log(jnp.sum(jnp.exp(z - m), axis=1, keepdims=True))
    out_ref[...] = z - m - lse


def _final(raw, dis8, ah, n, c):
    r = 2000
    grid = (n // r,)
    spec = pl.BlockSpec((r, c), lambda g: (g, 0))
    return pl.pallas_call(
        _final_body,
        grid=grid,
        in_specs=[spec, pl.BlockSpec((r, 8), lambda g: (g, 0)), spec],
        out_specs=spec,
        out_shape=jax.ShapeDtypeStruct((n, c), jnp.float32),
    )(raw, dis8, ah)


def kernel(x, edge_index, W, b):
    n, f = x.shape
    c = W.shape[1]
    e = edge_index.shape[1]
    nh = n // 2
    rt = nh // NSUB

    src = edge_index[0]
    dst = edge_index[1]

    quant = NSC * NSUB * GRP * B
    epad = (-e) % quant
    if epad:
        src = jnp.concatenate([src, jnp.zeros((epad,), jnp.int32)])
        dst = jnp.concatenate([dst, jnp.full((epad,), -1, jnp.int32)])
    nchunk = (e + epad) // B
    srcp = src.reshape(nchunk, B)
    dstp = dst.reshape(nchunk, B)

    dstl = _eprep(dstp, nh).reshape(NSC * nchunk, B)
    partials = _deg(dstp, n)
    b2 = b.reshape(1, c)
    u0, cc, aa, ah, dis8 = _prep(partials.T, x, W, b2, n, c)

    zr = jnp.zeros((rt, c), jnp.float32)
    nw = NSC * NSUB
    flen = rt * c
    cc32 = cc.reshape(nw, flen)
    aa32 = aa.reshape(nw, flen)
    u = u0
    raw = None
    for k in range(KHOPS):
        raw = _hop(u, srcp, dstl, zr, n, nh, rt, c)
        if k < KHOPS - 1:
            u = _upd(raw.reshape(nw, flen), cc32, aa32, nw, flen).reshape(n, c)
    return _final(raw.reshape(n, c), dis8, ah, n, c)


# speedup vs baseline: 1.1243x; 1.0306x over previous
import functools

import jax
import jax.numpy as jnp
from jax import lax
from jax.experimental import pallas as pl
from jax.experimental.pallas import tpu as pltpu
from jax.experimental.pallas import tpu_sc as plsc

ALPHA = 0.1
KHOPS = 10

B = 64
GRP = 2
NSC = 2
NSUB = 16


def _deg_body(n, nh, nchunk, dst_hbm, out_hbm, dstl_hbm,
              degv, didxb, obuf0, obuf1):
    c = lax.axis_index("c")
    s = lax.axis_index("s")
    w = c * NSUB + s
    zero16 = jnp.zeros((16,), jnp.float32)

    def zbody(i, _):
        degv[pl.ds(i * 16, 16)] = zero16
        return _

    lax.fori_loop(0, n // 16, zbody, None)

    ones16 = jnp.ones((16,), jnp.float32)
    gpt = nchunk // (NSC * NSUB) // GRP

    def gbody(g, _):
        g0 = (w * gpt + g) * GRP
        pltpu.sync_copy(dst_hbm.at[pl.ds(g0, GRP)], didxb)
        for jj in range(GRP):
            for ii in range(B // 16):
                idx = didxb[jj, pl.ds(ii * 16, 16)]
                plsc.addupdate_scatter(degv, [idx], ones16, mask=idx >= 0)
                l0 = jnp.where((idx >= 0) & (idx < nh), idx, nh)
                i1 = idx - nh
                l1 = jnp.where((i1 >= 0) & (i1 < nh), i1, nh)
                obuf0[jj, pl.ds(ii * 16, 16)] = l0
                obuf1[jj, pl.ds(ii * 16, 16)] = l1
        pltpu.sync_copy(obuf0, dstl_hbm.at[pl.ds(g0, GRP)])
        pltpu.sync_copy(obuf1, dstl_hbm.at[pl.ds(nchunk + g0, GRP)])
        return _

    lax.fori_loop(0, gpt, gbody, None)
    pltpu.sync_copy(degv, out_hbm.at[w])


def _deg(dstp, n, nh):
    nchunk = dstp.shape[0]
    mesh = plsc.VectorSubcoreMesh(core_axis_name="c", subcore_axis_name="s")
    return pl.kernel(
        functools.partial(_deg_body, n, nh, nchunk),
        out_type=[
            jax.ShapeDtypeStruct((NSC * NSUB, n), jnp.float32),
            jax.ShapeDtypeStruct((NSC * nchunk, B), jnp.int32),
        ],
        mesh=mesh,
        scratch_types=[
            pltpu.VMEM((n,), jnp.float32),
            pltpu.VMEM((GRP, B), jnp.int32),
            pltpu.VMEM((GRP, B), jnp.int32),
            pltpu.VMEM((GRP, B), jnp.int32),
        ],
        compiler_params=pltpu.CompilerParams(
            needs_layout_passes=False, use_tc_tiling_on_sc=False),
    )(dstp)


def _prep_body(c, part_ref, x_ref, w_ref, b_ref,
               u0_ref, cc_ref, aa_ref, ah_ref, dis_ref):
    deg = jnp.sum(part_ref[...], axis=1)
    h = jnp.dot(x_ref[...], w_ref[...],
                preferred_element_type=jnp.float32,
                precision=lax.Precision.HIGHEST) + b_ref[...]
    dis = jnp.where(deg > 0, lax.rsqrt(jnp.maximum(deg, 1e-12)), 0.0)
    d1 = dis[:, None]
    r = h.shape[0]
    u0_ref[...] = d1 * h
    aa_ref[...] = ALPHA * (d1 * h)
    cc_ref[...] = jnp.broadcast_to((1.0 - ALPHA) * d1 * d1, (r, c))
    ah_ref[...] = ALPHA * h
    dis_ref[...] = jnp.broadcast_to(d1, (r, 8))


def _prep(partials, x, w, b2, n, c):
    r = 1000
    f = x.shape[1]
    nw = partials.shape[1]
    grid = (n // r,)
    outs = [
        jax.ShapeDtypeStruct((n, c), jnp.float32),
        jax.ShapeDtypeStruct((n, c), jnp.float32),
        jax.ShapeDtypeStruct((n, c), jnp.float32),
        jax.ShapeDtypeStruct((n, c), jnp.float32),
        jax.ShapeDtypeStruct((n, 8), jnp.float32),
    ]
    return pl.pallas_call(
        functools.partial(_prep_body, c),
        grid=grid,
        in_specs=[
            pl.BlockSpec((r, nw), lambda g: (g, 0)),
            pl.BlockSpec((r, f), lambda g: (g, 0)),
            pl.BlockSpec((f, c), lambda g: (0, 0)),
            pl.BlockSpec((1, c), lambda g: (0, 0)),
        ],
        out_specs=[
            pl.BlockSpec((r, c), lambda g: (g, 0)),
            pl.BlockSpec((r, c), lambda g: (g, 0)),
            pl.BlockSpec((r, c), lambda g: (g, 0)),
            pl.BlockSpec((r, c), lambda g: (g, 0)),
            pl.BlockSpec((r, 8), lambda g: (g, 0)),
        ],
        out_shape=outs,
    )(partials, x, w, b2)


def _hop_body(n, nh, rt, nchunk, c_feat,
              u_hbm, src_hbm, dstl_hbm, zr_hbm, out_hbm,
              acc, sidx0, sidx1, didx0, didx1, rows0, rows1,
              gsem0, gsem1, ssem0, ssem1, is_s0, is_s1, is_d0, is_d1):
    c = lax.axis_index("c")
    s = lax.axis_index("s")

    pltpu.sync_copy(zr_hbm, acc.at[pl.ds(s * rt, rt)])

    @pl.when(s == NSUB - 1)
    def _():
        pltpu.sync_copy(zr_hbm.at[pl.ds(0, 8)], acc.at[pl.ds(nh, 8)])

    plsc.subcore_barrier()

    cpt = nchunk // NSUB
    base = s * cpt
    dbase = c * nchunk + base
    ngroups = cpt // GRP
    numu = ngroups // 2

    pltpu.sync_copy(src_hbm.at[pl.ds(base, GRP)], sidx0)
    pltpu.sync_copy(dstl_hbm.at[pl.ds(dbase, GRP)], didx0)

    def body(u_, _):
        k0 = u_ * 2 * GRP
        ib_s = pltpu.async_copy(
            src_hbm.at[pl.ds(base + k0 + GRP, GRP)], sidx1, is_s1)
        ib_d = pltpu.async_copy(
            dstl_hbm.at[pl.ds(dbase + k0 + GRP, GRP)], didx1, is_d1)
        g0 = pltpu.async_copy(u_hbm.at[sidx0.at[0]], rows0, gsem0)
        g1 = pltpu.async_copy(u_hbm.at[sidx0.at[1]], rows1, gsem1)
        g0.wait()
        s0 = pltpu.async_copy(rows0, acc.at[didx0.at[0]], ssem0, add=True)
        g1.wait()
        ib_s.wait()
        ib_d.wait()
        s0.wait()
        g2 = pltpu.async_copy(u_hbm.at[sidx1.at[0]], rows0, gsem0)
        s1 = pltpu.async_copy(rows1, acc.at[didx0.at[1]], ssem1, add=True)
        g2.wait()
        s1.wait()
        soff = jnp.minimum(base + k0 + 2 * GRP, nchunk - GRP)
        doff = jnp.minimum(dbase + k0 + 2 * GRP, NSC * nchunk - GRP)
        ia_s = pltpu.async_copy(src_hbm.at[pl.ds(soff, GRP)], sidx0, is_s0)
        ia_d = pltpu.async_copy(dstl_hbm.at[pl.ds(doff, GRP)], didx0, is_d0)
        g3 = pltpu.async_copy(u_hbm.at[sidx1.at[1]], rows1, gsem1)
        s2 = pltpu.async_copy(rows0, acc.at[didx1.at[0]], ssem0, add=True)
        g3.wait()
        s2.wait()
        s3 = pltpu.async_copy(rows1, acc.at[didx1.at[1]], ssem1, add=True)
        s3.wait()
        ia_s.wait()
        ia_d.wait()
        return _

    lax.fori_loop(0, numu, body, None)
    plsc.subcore_barrier()

    w = c * NSUB + s
    pltpu.sync_copy(acc.at[pl.ds(s * rt, rt)], out_hbm.at[w])


def _hop(u, srcp, dstl, zr, n, nh, rt, c):
    nchunk = srcp.shape[0]
    mesh = plsc.VectorSubcoreMesh(core_axis_name="c", subcore_axis_name="s")
    return pl.kernel(
        functools.partial(_hop_body, n, nh, rt, nchunk, c),
        out_type=jax.ShapeDtypeStruct((NSC * NSUB, rt, c), jnp.float32),
        mesh=mesh,
        scratch_types=[
            pltpu.VMEM_SHARED((nh + 8, c), jnp.float32),
            pltpu.VMEM((GRP, B), jnp.int32),
            pltpu.VMEM((GRP, B), jnp.int32),
            pltpu.VMEM((GRP, B), jnp.int32),
            pltpu.VMEM((GRP, B), jnp.int32),
            pltpu.VMEM((B, c), jnp.float32),
            pltpu.VMEM((B, c), jnp.float32),
            pltpu.SemaphoreType.DMA,
            pltpu.SemaphoreType.DMA,
            pltpu.SemaphoreType.DMA,
            pltpu.SemaphoreType.DMA,
            pltpu.SemaphoreType.DMA,
            pltpu.SemaphoreType.DMA,
            pltpu.SemaphoreType.DMA,
            pltpu.SemaphoreType.DMA,
        ],
        compiler_params=pltpu.CompilerParams(
            needs_layout_passes=False, use_tc_tiling_on_sc=False),
    )(u, srcp, dstl, zr)


def _upd_body(flen, raw_hbm, cc_hbm, aa_hbm, out_hbm, rbuf, cbuf, abuf):
    c = lax.axis_index("c")
    s = lax.axis_index("s")
    w = c * NSUB + s
    ch = rbuf.shape[0]
    nfull = flen // ch
    rem = flen - nfull * ch

    def run_chunk(off, size):
        pltpu.sync_copy(raw_hbm.at[w, pl.ds(off, size)], rbuf.at[pl.ds(0, size)])
        pltpu.sync_copy(cc_hbm.at[w, pl.ds(off, size)], cbuf.at[pl.ds(0, size)])
        pltpu.sync_copy(aa_hbm.at[w, pl.ds(off, size)], abuf.at[pl.ds(0, size)])

        def vbody(i, t):
            p = i * 16
            rbuf[pl.ds(p, 16)] = (rbuf[pl.ds(p, 16)] * cbuf[pl.ds(p, 16)]
                                  + abuf[pl.ds(p, 16)])
            return t

        if size % 16:
            p = size - 16
            tailv = (rbuf[pl.ds(p, 16)] * cbuf[pl.ds(p, 16)]
                     + abuf[pl.ds(p, 16)])
            tailv = lax.fori_loop(0, size // 16, vbody, tailv)
            rbuf[pl.ds(p, 16)] = tailv
        else:
            lax.fori_loop(0, size // 16, vbody, jnp.zeros((16,), jnp.float32))
        pltpu.sync_copy(rbuf.at[pl.ds(0, size)], out_hbm.at[w, pl.ds(off, size)])

    def cbody(j, _):
        run_chunk(j * ch, ch)
        return _

    lax.fori_loop(0, nfull, cbody, None)
    if rem:
        run_chunk(nfull * ch, rem)


def _upd(raw32, cc32, aa32, nw, flen):
    ch = 24576
    mesh = plsc.VectorSubcoreMesh(core_axis_name="c", subcore_axis_name="s")
    return pl.kernel(
        functools.partial(_upd_body, flen),
        out_type=jax.ShapeDtypeStruct((nw, flen), jnp.float32),
        mesh=mesh,
        scratch_types=[
            pltpu.VMEM((ch,), jnp.float32),
            pltpu.VMEM((ch,), jnp.float32),
            pltpu.VMEM((ch,), jnp.float32),
        ],
        compiler_params=pltpu.CompilerParams(
            needs_layout_passes=False, use_tc_tiling_on_sc=False),
    )(raw32, cc32, aa32)


def _final_body(raw_ref, dis_ref, ah_ref, out_ref):
    z = (1.0 - ALPHA) * dis_ref[:, 0:1] * raw_ref[...] + ah_ref[...]
    m = jnp.max(z, axis=1, keepdims=True)
    lse = jnp.log(jnp.sum(jnp.exp(z - m), axis=1, keepdims=True))
    out_ref[...] = z - m - lse


def _final(raw, dis8, ah, n, c):
    r = 2000
    grid = (n // r,)
    spec = pl.BlockSpec((r, c), lambda g: (g, 0))
    return pl.pallas_call(
        _final_body,
        grid=grid,
        in_specs=[spec, pl.BlockSpec((r, 8), lambda g: (g, 0)), spec],
        out_specs=spec,
        out_shape=jax.ShapeDtypeStruct((n, c), jnp.float32),
    )(raw, dis8, ah)


def kernel(x, edge_index, W, b):
    n, f = x.shape
    c = W.shape[1]
    e = edge_index.shape[1]
    nh = n // 2
    rt = nh // NSUB

    src = edge_index[0]
    dst = edge_index[1]

    quant = NSC * NSUB * GRP * B
    epad = (-e) % quant
    if epad:
        src = jnp.concatenate([src, jnp.zeros((epad,), jnp.int32)])
        dst = jnp.concatenate([dst, jnp.full((epad,), -1, jnp.int32)])
    nchunk = (e + epad) // B
    srcp = src.reshape(nchunk, B)
    dstp = dst.reshape(nchunk, B)

    partials, dstl = _deg(dstp, n, nh)
    b2 = b.reshape(1, c)
    u0, cc, aa, ah, dis8 = _prep(partials.T, x, W, b2, n, c)

    zr = jnp.zeros((rt, c), jnp.float32)
    nw = NSC * NSUB
    flen = rt * c
    cc32 = cc.reshape(nw, flen)
    aa32 = aa.reshape(nw, flen)
    u = u0
    raw = None
    for k in range(KHOPS):
        raw = _hop(u, srcp, dstl, zr, n, nh, rt, c)
        if k < KHOPS - 1:
            u = _upd(raw.reshape(nw, flen), cc32, aa32, nw, flen).reshape(n, c)
    return _final(raw.reshape(n, c), dis8, ah, n, c)
